# Initial kernel scaffold; baseline (speedup 1.0000x reference)
import jax
import jax.numpy as jnp
from jax.experimental import pallas as pl


def kernel(user_ids, video_ids, user_table, video_table, uW1, ub1, uW2, ub2, vW1, vb1, vW2, vb2):
    raise NotImplementedError("write your pallas kernel here")


# speedup vs baseline: 2.6751x; 2.6751x over previous
import functools

import jax
import jax.numpy as jnp
from jax import lax
from jax.experimental import pallas as pl
from jax.experimental.pallas import tpu as pltpu
from jax.experimental.pallas import tpu_sc as plsc

VOCAB = 100000
B = 16384
D = 128
H = 256

NC = 2
NS = 16
NW = NC * NS
CHUNK = 128
ROWS_PER_W = 2 * B // NW
CH_PER_TABLE = (B // NW) // CHUNK


def _sc_gather(ids2d, user_table, video_table):
    mesh = plsc.VectorSubcoreMesh(core_axis_name="c", subcore_axis_name="s")

    @functools.partial(
        pl.kernel,
        out_type=jax.ShapeDtypeStruct((2 * B, D), jnp.float32),
        mesh=mesh,
        scratch_types=[
            pltpu.VMEM((2 * CH_PER_TABLE, CHUNK), jnp.int32),
            pltpu.VMEM((CHUNK, D), jnp.float32),
            pltpu.VMEM((CHUNK, D), jnp.float32),
            pltpu.SemaphoreType.DMA,
            pltpu.SemaphoreType.DMA,
        ],
    )
    def k(ids_hbm, utab_hbm, vtab_hbm, out_hbm, idx_v, rows_a, rows_b, sem_a, sem_b):
        wid = lax.axis_index("s") * NC + lax.axis_index("c")
        u_row0 = wid * CH_PER_TABLE
        v_row0 = B // CHUNK + wid * CH_PER_TABLE
        pltpu.sync_copy(ids_hbm.at[pl.ds(u_row0, CH_PER_TABLE)],
                        idx_v.at[pl.ds(0, CH_PER_TABLE)])
        pltpu.sync_copy(ids_hbm.at[pl.ds(v_row0, CH_PER_TABLE)],
                        idx_v.at[pl.ds(CH_PER_TABLE, CH_PER_TABLE)])

        bufs = (rows_a, rows_b)
        sems = (sem_a, sem_b)
        tabs = (utab_hbm, vtab_hbm)
        total = 2 * CH_PER_TABLE
        copies = []
        for j in range(total):
            t = j // CH_PER_TABLE
            cp = pltpu.make_async_copy(
                tabs[t].at[idx_v.at[j]], bufs[j % 2], sems[j % 2])
            cp.start()
            if j > 0:
                copies[j - 1].wait()
                jp = j - 1
                tp = jp // CH_PER_TABLE
                base = tp * B + wid * (B // NW) + (jp % CH_PER_TABLE) * CHUNK
                pltpu.sync_copy(bufs[jp % 2], out_hbm.at[pl.ds(base, CHUNK)])
            copies.append(cp)
        copies[total - 1].wait()
        jp = total - 1
        tp = jp // CH_PER_TABLE
        base = tp * B + wid * (B // NW) + (jp % CH_PER_TABLE) * CHUNK
        pltpu.sync_copy(bufs[jp % 2], out_hbm.at[pl.ds(base, CHUNK)])

    return k(ids2d, user_table, video_table)


BLK = 1024


def _tower_body(emb_ref, w1_ref, b1_ref, w2_ref, b2_ref, out_ref):
    x = emb_ref[...]
    h = jnp.dot(x, w1_ref[0], preferred_element_type=jnp.float32)
    h = jnp.maximum(h + b1_ref[0], 0.0)
    y = jnp.dot(h, w2_ref[0], preferred_element_type=jnp.float32)
    y = y + b2_ref[0]
    ss = jnp.sum(y * y, axis=1, keepdims=True)
    out_ref[0] = y * lax.rsqrt(jnp.maximum(ss, 1e-12))


def _tc_towers(emb, W1, b1, W2, b2):
    nblk = B // BLK
    return pl.pallas_call(
        _tower_body,
        grid=(2, nblk),
        in_specs=[
            pl.BlockSpec((BLK, D), lambda t, i, n=nblk: (t * n + i, 0)),
            pl.BlockSpec((1, D, H), lambda t, i: (t, 0, 0)),
            pl.BlockSpec((1, 1, H), lambda t, i: (t, 0, 0)),
            pl.BlockSpec((1, H, D), lambda t, i: (t, 0, 0)),
            pl.BlockSpec((1, 1, D), lambda t, i: (t, 0, 0)),
        ],
        out_specs=pl.BlockSpec((1, BLK, D), lambda t, i: (t, i, 0)),
        out_shape=jax.ShapeDtypeStruct((2, B, D), jnp.float32),
    )(emb, W1, b1, W2, b2)


def kernel(user_ids, video_ids, user_table, video_table,
           uW1, ub1, uW2, ub2, vW1, vb1, vW2, vb2):
    ids2d = jnp.concatenate([
        user_ids.astype(jnp.int32), video_ids.astype(jnp.int32)
    ]).reshape(2 * B // CHUNK, CHUNK)
    emb = _sc_gather(ids2d, user_table, video_table)
    W1 = jnp.stack([uW1, vW1])
    b1 = jnp.stack([ub1, vb1])[:, None, :]
    W2 = jnp.stack([uW2, vW2])
    b2 = jnp.stack([ub2, vb2])[:, None, :]
    return _tc_towers(emb, W1, b1, W2, b2)


# speedup vs baseline: 2.6776x; 1.0010x over previous
import functools

import jax
import jax.numpy as jnp
from jax import lax
from jax.experimental import pallas as pl
from jax.experimental.pallas import tpu as pltpu
from jax.experimental.pallas import tpu_sc as plsc

VOCAB = 100000
B = 16384
D = 128
H = 256

NC = 2
NS = 16
NW = NC * NS
CHUNK = 128
ROWS_PER_W = 2 * B // NW
CH_PER_TABLE = (B // NW) // CHUNK


def _sc_gather(ids2d, user_table, video_table):
    mesh = plsc.VectorSubcoreMesh(core_axis_name="c", subcore_axis_name="s")

    @functools.partial(
        pl.kernel,
        out_type=jax.ShapeDtypeStruct((2 * B, D), jnp.float32),
        mesh=mesh,
        scratch_types=[
            pltpu.VMEM((2 * CH_PER_TABLE, CHUNK), jnp.int32),
            pltpu.VMEM((CHUNK, D), jnp.float32),
            pltpu.VMEM((CHUNK, D), jnp.float32),
            pltpu.SemaphoreType.DMA,
            pltpu.SemaphoreType.DMA,
        ],
    )
    def k(ids_hbm, utab_hbm, vtab_hbm, out_hbm, idx_v, rows_a, rows_b, sem_a, sem_b):
        wid = lax.axis_index("s") * NC + lax.axis_index("c")
        u_row0 = wid * CH_PER_TABLE
        v_row0 = B // CHUNK + wid * CH_PER_TABLE
        pltpu.sync_copy(ids_hbm.at[pl.ds(u_row0, CH_PER_TABLE)],
                        idx_v.at[pl.ds(0, CH_PER_TABLE)])
        pltpu.sync_copy(ids_hbm.at[pl.ds(v_row0, CH_PER_TABLE)],
                        idx_v.at[pl.ds(CH_PER_TABLE, CH_PER_TABLE)])

        bufs = (rows_a, rows_b)
        sems = (sem_a, sem_b)
        tabs = (utab_hbm, vtab_hbm)
        total = 2 * CH_PER_TABLE
        copies = []
        for j in range(total):
            t = j // CH_PER_TABLE
            cp = pltpu.make_async_copy(
                tabs[t].at[idx_v.at[j]], bufs[j % 2], sems[j % 2])
            cp.start()
            if j > 0:
                copies[j - 1].wait()
                jp = j - 1
                tp = jp // CH_PER_TABLE
                base = tp * B + wid * (B // NW) + (jp % CH_PER_TABLE) * CHUNK
                pltpu.sync_copy(bufs[jp % 2], out_hbm.at[pl.ds(base, CHUNK)])
            copies.append(cp)
        copies[total - 1].wait()
        jp = total - 1
        tp = jp // CH_PER_TABLE
        base = tp * B + wid * (B // NW) + (jp % CH_PER_TABLE) * CHUNK
        pltpu.sync_copy(bufs[jp % 2], out_hbm.at[pl.ds(base, CHUNK)])

    return k(ids2d, user_table, video_table)


BLK = 1024


def _tower_body(emb_ref, w1_ref, b1_ref, w2_ref, b2_ref, out_ref):
    x = emb_ref[...].astype(jnp.bfloat16)
    h = jnp.dot(x, w1_ref[0], preferred_element_type=jnp.float32)
    h = jnp.maximum(h + b1_ref[0], 0.0).astype(jnp.bfloat16)
    y = jnp.dot(h, w2_ref[0], preferred_element_type=jnp.float32)
    y = y + b2_ref[0]
    ss = jnp.sum(y * y, axis=1, keepdims=True)
    out_ref[0] = y * lax.rsqrt(jnp.maximum(ss, 1e-12))


def _tc_towers(emb, W1, b1, W2, b2):
    nblk = B // BLK
    return pl.pallas_call(
        _tower_body,
        grid=(2, nblk),
        in_specs=[
            pl.BlockSpec((BLK, D), lambda t, i, n=nblk: (t * n + i, 0)),
            pl.BlockSpec((1, D, H), lambda t, i: (t, 0, 0)),
            pl.BlockSpec((1, 1, H), lambda t, i: (t, 0, 0)),
            pl.BlockSpec((1, H, D), lambda t, i: (t, 0, 0)),
            pl.BlockSpec((1, 1, D), lambda t, i: (t, 0, 0)),
        ],
        out_specs=pl.BlockSpec((1, BLK, D), lambda t, i: (t, i, 0)),
        out_shape=jax.ShapeDtypeStruct((2, B, D), jnp.float32),
    )(emb, W1, b1, W2, b2)


def kernel(user_ids, video_ids, user_table, video_table,
           uW1, ub1, uW2, ub2, vW1, vb1, vW2, vb2):
    ids2d = jnp.concatenate([
        user_ids.astype(jnp.int32), video_ids.astype(jnp.int32)
    ]).reshape(2 * B // CHUNK, CHUNK)
    emb = _sc_gather(ids2d, user_table, video_table)
    W1 = jnp.stack([uW1, vW1]).astype(jnp.bfloat16)
    b1 = jnp.stack([ub1, vb1])[:, None, :]
    W2 = jnp.stack([uW2, vW2]).astype(jnp.bfloat16)
    b2 = jnp.stack([ub2, vb2])[:, None, :]
    return _tc_towers(emb, W1, b1, W2, b2)


# speedup vs baseline: 3.1567x; 1.1789x over previous
import functools

import jax
import jax.numpy as jnp
from jax import lax
from jax.experimental import pallas as pl
from jax.experimental.pallas import tpu as pltpu
from jax.experimental.pallas import tpu_sc as plsc

VOCAB = 100000
B = 16384
D = 128
H = 256

NC = 2
NS = 16
NW = NC * NS
CHUNK = 128
CH_PER_TABLE = (B // NW) // CHUNK


def _sc_gather(uids2d, vids2d, user_table, video_table):
    mesh = plsc.VectorSubcoreMesh(core_axis_name="c", subcore_axis_name="s")

    @functools.partial(
        pl.kernel,
        out_type=jax.ShapeDtypeStruct((2 * B, D), jnp.float32),
        mesh=mesh,
        scratch_types=[
            pltpu.VMEM((2 * CH_PER_TABLE, CHUNK), jnp.int32),
            pltpu.VMEM((CHUNK, D), jnp.float32),
            pltpu.VMEM((CHUNK, D), jnp.float32),
            pltpu.SemaphoreType.DMA,
            pltpu.SemaphoreType.DMA,
        ],
    )
    def k(uids_hbm, vids_hbm, utab_hbm, vtab_hbm, out_hbm,
          idx_v, rows_a, rows_b, sem_a, sem_b):
        wid = lax.axis_index("s") * NC + lax.axis_index("c")
        row0 = wid * CH_PER_TABLE
        pltpu.sync_copy(uids_hbm.at[pl.ds(row0, CH_PER_TABLE)],
                        idx_v.at[pl.ds(0, CH_PER_TABLE)])
        pltpu.sync_copy(vids_hbm.at[pl.ds(row0, CH_PER_TABLE)],
                        idx_v.at[pl.ds(CH_PER_TABLE, CH_PER_TABLE)])

        bufs = (rows_a, rows_b)
        sems = (sem_a, sem_b)
        tabs = (utab_hbm, vtab_hbm)
        total = 2 * CH_PER_TABLE
        copies = []
        for j in range(total + 1):
            if j < total:
                t = j // CH_PER_TABLE
                cp = pltpu.make_async_copy(
                    tabs[t].at[idx_v.at[j]], bufs[j % 2], sems[j % 2])
                cp.start()
                copies.append(cp)
            if j > 0:
                jp = j - 1
                copies[jp].wait()
                tp = jp // CH_PER_TABLE
                base = tp * B + wid * (B // NW) + (jp % CH_PER_TABLE) * CHUNK
                pltpu.sync_copy(bufs[jp % 2], out_hbm.at[pl.ds(base, CHUNK)])

    return k(uids2d, vids2d, user_table, video_table)


BLK = 1024


def _towers_body(xu_ref, xv_ref,
                 uw1_ref, ub1_ref, uw2_ref, ub2_ref,
                 vw1_ref, vb1_ref, vw2_ref, vb2_ref, out_ref):
    for t, (x_ref, w1_ref, b1_ref, w2_ref, b2_ref) in enumerate((
            (xu_ref, uw1_ref, ub1_ref, uw2_ref, ub2_ref),
            (xv_ref, vw1_ref, vb1_ref, vw2_ref, vb2_ref))):
        x = x_ref[...].astype(jnp.bfloat16)
        h = jnp.dot(x, w1_ref[...], preferred_element_type=jnp.float32)
        h = jnp.maximum(h + b1_ref[...], 0.0).astype(jnp.bfloat16)
        y = jnp.dot(h, w2_ref[...], preferred_element_type=jnp.float32)
        y = y + b2_ref[...]
        ss = jnp.sum(y * y, axis=1, keepdims=True)
        out_ref[t] = y * lax.rsqrt(jnp.maximum(ss, 1e-12))


def _tc_towers(emb, uW1, ub1, uW2, ub2, vW1, vb1, vW2, vb2):
    nblk = B // BLK
    wspec1 = pl.BlockSpec((D, H), lambda i: (0, 0))
    bspec1 = pl.BlockSpec((1, H), lambda i: (0, 0))
    wspec2 = pl.BlockSpec((H, D), lambda i: (0, 0))
    bspec2 = pl.BlockSpec((1, D), lambda i: (0, 0))
    return pl.pallas_call(
        _towers_body,
        grid=(nblk,),
        in_specs=[
            pl.BlockSpec((BLK, D), lambda i: (i, 0)),
            pl.BlockSpec((BLK, D), lambda i, n=nblk: (n + i, 0)),
            wspec1, bspec1, wspec2, bspec2,
            wspec1, bspec1, wspec2, bspec2,
        ],
        out_specs=pl.BlockSpec((2, BLK, D), lambda i: (0, i, 0)),
        out_shape=jax.ShapeDtypeStruct((2, B, D), jnp.float32),
    )(emb, emb, uW1, ub1, uW2, ub2, vW1, vb1, vW2, vb2)


def kernel(user_ids, video_ids, user_table, video_table,
           uW1, ub1, uW2, ub2, vW1, vb1, vW2, vb2):
    uids2d = user_ids.astype(jnp.int32).reshape(B // CHUNK, CHUNK)
    vids2d = video_ids.astype(jnp.int32).reshape(B // CHUNK, CHUNK)
    emb = _sc_gather(uids2d, vids2d, user_table, video_table)
    return _tc_towers(
        emb,
        uW1.astype(jnp.bfloat16), ub1[None, :],
        uW2.astype(jnp.bfloat16), ub2[None, :],
        vW1.astype(jnp.bfloat16), vb1[None, :],
        vW2.astype(jnp.bfloat16), vb2[None, :],
    )


# speedup vs baseline: 3.1802x; 1.0074x over previous
import functools

import jax
import jax.numpy as jnp
from jax import lax
from jax.experimental import pallas as pl
from jax.experimental.pallas import tpu as pltpu
from jax.experimental.pallas import tpu_sc as plsc

VOCAB = 100000
B = 16384
D = 128
H = 256

NC = 2
NS = 16
NW = NC * NS
CHUNK = 128
CH_PER_TABLE = (B // NW) // CHUNK
NBUF = 4


def _sc_gather(uids2d, vids2d, user_table, video_table):
    mesh = plsc.VectorSubcoreMesh(core_axis_name="c", subcore_axis_name="s")

    @functools.partial(
        pl.kernel,
        out_type=jax.ShapeDtypeStruct((2 * B, D), jnp.float32),
        mesh=mesh,
        scratch_types=[
            pltpu.VMEM((2 * CH_PER_TABLE, CHUNK), jnp.int32),
            [pltpu.VMEM((CHUNK, D), jnp.float32) for _ in range(NBUF)],
            [pltpu.SemaphoreType.DMA for _ in range(NBUF)],
            [pltpu.SemaphoreType.DMA for _ in range(NBUF)],
        ],
    )
    def k(uids_hbm, vids_hbm, utab_hbm, vtab_hbm, out_hbm,
          idx_v, bufs, sems, wsems):
        wid = lax.axis_index("s") * NC + lax.axis_index("c")
        row0 = wid * CH_PER_TABLE
        pltpu.sync_copy(uids_hbm.at[pl.ds(row0, CH_PER_TABLE)],
                        idx_v.at[pl.ds(0, CH_PER_TABLE)])
        pltpu.sync_copy(vids_hbm.at[pl.ds(row0, CH_PER_TABLE)],
                        idx_v.at[pl.ds(CH_PER_TABLE, CH_PER_TABLE)])

        tabs = (utab_hbm, vtab_hbm)
        total = 2 * CH_PER_TABLE
        gathers = [None] * total
        scatters = [None] * total
        for j in range(total):
            b = j % NBUF
            if j >= NBUF:
                scatters[j - NBUF].wait()
            cp = pltpu.make_async_copy(tabs[j // CH_PER_TABLE].at[idx_v.at[j]],
                                       bufs[b], sems[b])
            cp.start()
            gathers[j] = cp
            if j > 0:
                jp = j - 1
                gathers[jp].wait()
                tp = jp // CH_PER_TABLE
                base = tp * B + wid * (B // NW) + (jp % CH_PER_TABLE) * CHUNK
                sc = pltpu.make_async_copy(
                    bufs[jp % NBUF], out_hbm.at[pl.ds(base, CHUNK)], wsems[jp % NBUF])
                sc.start()
                scatters[jp] = sc
        jp = total - 1
        gathers[jp].wait()
        base = B + wid * (B // NW) + (jp % CH_PER_TABLE) * CHUNK
        sc = pltpu.make_async_copy(
            bufs[jp % NBUF], out_hbm.at[pl.ds(base, CHUNK)], wsems[jp % NBUF])
        sc.start()
        scatters[jp] = sc
        for j in range(total - NBUF, total):
            scatters[j].wait()

    return k(uids2d, vids2d, user_table, video_table)


BLK = 1024


def _towers_body(xu_ref, xv_ref,
                 uw1_ref, ub1_ref, uw2_ref, ub2_ref,
                 vw1_ref, vb1_ref, vw2_ref, vb2_ref, out_ref):
    for t, (x_ref, w1_ref, b1_ref, w2_ref, b2_ref) in enumerate((
            (xu_ref, uw1_ref, ub1_ref, uw2_ref, ub2_ref),
            (xv_ref, vw1_ref, vb1_ref, vw2_ref, vb2_ref))):
        x = x_ref[...].astype(jnp.bfloat16)
        h = jnp.dot(x, w1_ref[...].astype(jnp.bfloat16),
                    preferred_element_type=jnp.float32)
        h = jnp.maximum(h + b1_ref[...], 0.0).astype(jnp.bfloat16)
        y = jnp.dot(h, w2_ref[...].astype(jnp.bfloat16),
                    preferred_element_type=jnp.float32)
        y = y + b2_ref[...]
        ss = jnp.sum(y * y, axis=1, keepdims=True)
        out_ref[t] = y * lax.rsqrt(jnp.maximum(ss, 1e-12))


def _tc_towers(emb, uW1, ub1, uW2, ub2, vW1, vb1, vW2, vb2):
    nblk = B // BLK
    wspec1 = pl.BlockSpec((D, H), lambda i: (0, 0))
    bspec1 = pl.BlockSpec((1, H), lambda i: (0, 0))
    wspec2 = pl.BlockSpec((H, D), lambda i: (0, 0))
    bspec2 = pl.BlockSpec((1, D), lambda i: (0, 0))
    return pl.pallas_call(
        _towers_body,
        grid=(nblk,),
        in_specs=[
            pl.BlockSpec((BLK, D), lambda i: (i, 0)),
            pl.BlockSpec((BLK, D), lambda i, n=nblk: (n + i, 0)),
            wspec1, bspec1, wspec2, bspec2,
            wspec1, bspec1, wspec2, bspec2,
        ],
        out_specs=pl.BlockSpec((2, BLK, D), lambda i: (0, i, 0)),
        out_shape=jax.ShapeDtypeStruct((2, B, D), jnp.float32),
    )(emb, emb, uW1, ub1, uW2, ub2, vW1, vb1, vW2, vb2)


def kernel(user_ids, video_ids, user_table, video_table,
           uW1, ub1, uW2, ub2, vW1, vb1, vW2, vb2):
    uids2d = user_ids.astype(jnp.int32).reshape(B // CHUNK, CHUNK)
    vids2d = video_ids.astype(jnp.int32).reshape(B // CHUNK, CHUNK)
    emb = _sc_gather(uids2d, vids2d, user_table, video_table)
    return _tc_towers(
        emb,
        uW1, ub1[None, :], uW2, ub2[None, :],
        vW1, vb1[None, :], vW2, vb2[None, :],
    )


# speedup vs baseline: 3.5173x; 1.1060x over previous
import functools

import jax
import jax.numpy as jnp
from jax import lax
from jax.experimental import pallas as pl
from jax.experimental.pallas import tpu as pltpu
from jax.experimental.pallas import tpu_sc as plsc

VOCAB = 100000
B = 16384
D = 128
H = 256

NC = 2
NS = 16
NW = NC * NS
CHUNK = 128
CH_PER_TABLE = (B // NW) // CHUNK
NBUF = 4


def _sc_gather(uids2d, vids2d, user_table, video_table):
    mesh = plsc.VectorSubcoreMesh(core_axis_name="c", subcore_axis_name="s")

    @functools.partial(
        pl.kernel,
        out_type=jax.ShapeDtypeStruct((2 * B, D), jnp.float32),
        mesh=mesh,
        scratch_types=[
            pltpu.VMEM((2 * CH_PER_TABLE, CHUNK), jnp.int32),
            [pltpu.VMEM((CHUNK, D), jnp.float32) for _ in range(NBUF)],
            [pltpu.SemaphoreType.DMA for _ in range(NBUF)],
            [pltpu.SemaphoreType.DMA for _ in range(NBUF)],
        ],
    )
    def k(uids_hbm, vids_hbm, utab_hbm, vtab_hbm, out_hbm,
          idx_v, bufs, sems, wsems):
        wid = lax.axis_index("s") * NC + lax.axis_index("c")
        row0 = wid * CH_PER_TABLE
        icp_u = pltpu.make_async_copy(uids_hbm.at[pl.ds(row0, CH_PER_TABLE)],
                                      idx_v.at[pl.ds(0, CH_PER_TABLE)], wsems[0])
        icp_v = pltpu.make_async_copy(vids_hbm.at[pl.ds(row0, CH_PER_TABLE)],
                                      idx_v.at[pl.ds(CH_PER_TABLE, CH_PER_TABLE)],
                                      wsems[1])
        icp_u.start()
        icp_v.start()
        icp_u.wait()
        icp_v.wait()

        tabs = (utab_hbm, vtab_hbm)
        total = 2 * CH_PER_TABLE
        gathers = [None] * total
        scatters = [None] * total
        for j in range(total):
            b = j % NBUF
            if j >= NBUF:
                scatters[j - NBUF].wait()
            cp = pltpu.make_async_copy(tabs[j // CH_PER_TABLE].at[idx_v.at[j]],
                                       bufs[b], sems[b])
            cp.start()
            gathers[j] = cp
            if j > 0:
                jp = j - 1
                gathers[jp].wait()
                tp = jp // CH_PER_TABLE
                base = tp * B + wid * (B // NW) + (jp % CH_PER_TABLE) * CHUNK
                sc = pltpu.make_async_copy(
                    bufs[jp % NBUF], out_hbm.at[pl.ds(base, CHUNK)], wsems[jp % NBUF])
                sc.start()
                scatters[jp] = sc
        jp = total - 1
        gathers[jp].wait()
        base = B + wid * (B // NW) + (jp % CH_PER_TABLE) * CHUNK
        sc = pltpu.make_async_copy(
            bufs[jp % NBUF], out_hbm.at[pl.ds(base, CHUNK)], wsems[jp % NBUF])
        sc.start()
        scatters[jp] = sc
        for j in range(total - NBUF, total):
            scatters[j].wait()

    return k(uids2d, vids2d, user_table, video_table)


BLK = 2048


def _towers_body(xu_ref, xv_ref,
                 uw1_ref, ub1_ref, uw2_ref, ub2_ref,
                 vw1_ref, vb1_ref, vw2_ref, vb2_ref, out_ref):
    for t, (x_ref, w1_ref, b1_ref, w2_ref, b2_ref) in enumerate((
            (xu_ref, uw1_ref, ub1_ref, uw2_ref, ub2_ref),
            (xv_ref, vw1_ref, vb1_ref, vw2_ref, vb2_ref))):
        x = x_ref[...].astype(jnp.bfloat16)
        h = jnp.dot(x, w1_ref[...].astype(jnp.bfloat16),
                    preferred_element_type=jnp.float32)
        h = jnp.maximum(h + b1_ref[...], 0.0).astype(jnp.bfloat16)
        y = jnp.dot(h, w2_ref[...].astype(jnp.bfloat16),
                    preferred_element_type=jnp.float32)
        y = y + b2_ref[...]
        ss = jnp.sum(y * y, axis=1, keepdims=True)
        out_ref[t] = y * lax.rsqrt(jnp.maximum(ss, 1e-12))


def _tc_towers(emb, uW1, ub1, uW2, ub2, vW1, vb1, vW2, vb2):
    nblk = B // BLK
    wspec1 = pl.BlockSpec((D, H), lambda i: (0, 0))
    bspec1 = pl.BlockSpec((1, H), lambda i: (0, 0))
    wspec2 = pl.BlockSpec((H, D), lambda i: (0, 0))
    bspec2 = pl.BlockSpec((1, D), lambda i: (0, 0))
    return pl.pallas_call(
        _towers_body,
        grid=(nblk,),
        in_specs=[
            pl.BlockSpec((BLK, D), lambda i: (i, 0)),
            pl.BlockSpec((BLK, D), lambda i, n=nblk: (n + i, 0)),
            wspec1, bspec1, wspec2, bspec2,
            wspec1, bspec1, wspec2, bspec2,
        ],
        out_specs=pl.BlockSpec((2, BLK, D), lambda i: (0, i, 0)),
        out_shape=jax.ShapeDtypeStruct((2, B, D), jnp.float32),
    )(emb, emb, uW1, ub1, uW2, ub2, vW1, vb1, vW2, vb2)


def kernel(user_ids, video_ids, user_table, video_table,
           uW1, ub1, uW2, ub2, vW1, vb1, vW2, vb2):
    uids2d = user_ids.astype(jnp.int32).reshape(B // CHUNK, CHUNK)
    vids2d = video_ids.astype(jnp.int32).reshape(B // CHUNK, CHUNK)
    emb = _sc_gather(uids2d, vids2d, user_table, video_table)
    return _tc_towers(
        emb,
        uW1, ub1[None, :], uW2, ub2[None, :],
        vW1, vb1[None, :], vW2, vb2[None, :],
    )


# speedup vs baseline: 3.6571x; 1.0397x over previous
import functools

import jax
import jax.numpy as jnp
from jax import lax
from jax.experimental import pallas as pl
from jax.experimental.pallas import tpu as pltpu
from jax.experimental.pallas import tpu_sc as plsc

VOCAB = 100000
B = 16384
D = 128
H = 256

NC = 2
NS = 16
NW = NC * NS
CHUNK = 128
CH_PER_TABLE = (B // NW) // CHUNK
NBUF = 4


def _sc_gather(uids2d, vids2d, user_table, video_table):
    mesh = plsc.VectorSubcoreMesh(core_axis_name="c", subcore_axis_name="s")

    @functools.partial(
        pl.kernel,
        out_type=jax.ShapeDtypeStruct((2 * B, D), jnp.float32),
        mesh=mesh,
        scratch_types=[
            pltpu.VMEM((2 * CH_PER_TABLE, CHUNK), jnp.int32),
            [pltpu.VMEM((CHUNK, D), jnp.float32) for _ in range(NBUF)],
            [pltpu.SemaphoreType.DMA for _ in range(NBUF)],
            [pltpu.SemaphoreType.DMA for _ in range(NBUF)],
        ],
    )
    def k(uids_hbm, vids_hbm, utab_hbm, vtab_hbm, out_hbm,
          idx_v, bufs, sems, wsems):
        wid = lax.axis_index("s") * NC + lax.axis_index("c")
        row0 = wid * CH_PER_TABLE
        icp_u = pltpu.make_async_copy(uids_hbm.at[pl.ds(row0, CH_PER_TABLE)],
                                      idx_v.at[pl.ds(0, CH_PER_TABLE)], wsems[0])
        icp_v = pltpu.make_async_copy(vids_hbm.at[pl.ds(row0, CH_PER_TABLE)],
                                      idx_v.at[pl.ds(CH_PER_TABLE, CH_PER_TABLE)],
                                      wsems[1])
        icp_u.start()
        icp_v.start()
        icp_u.wait()
        icp_v.wait()

        tabs = (utab_hbm, vtab_hbm)
        total = 2 * CH_PER_TABLE
        gathers = [None] * total
        scatters = [None] * total
        for j in range(total):
            b = j % NBUF
            if j >= NBUF:
                scatters[j - NBUF].wait()
            cp = pltpu.make_async_copy(tabs[j // CH_PER_TABLE].at[idx_v.at[j]],
                                       bufs[b], sems[b])
            cp.start()
            gathers[j] = cp
            if j > 0:
                jp = j - 1
                gathers[jp].wait()
                tp = jp // CH_PER_TABLE
                base = tp * B + wid * (B // NW) + (jp % CH_PER_TABLE) * CHUNK
                sc = pltpu.make_async_copy(
                    bufs[jp % NBUF], out_hbm.at[pl.ds(base, CHUNK)], wsems[jp % NBUF])
                sc.start()
                scatters[jp] = sc
        jp = total - 1
        gathers[jp].wait()
        base = B + wid * (B // NW) + (jp % CH_PER_TABLE) * CHUNK
        sc = pltpu.make_async_copy(
            bufs[jp % NBUF], out_hbm.at[pl.ds(base, CHUNK)], wsems[jp % NBUF])
        sc.start()
        scatters[jp] = sc
        for j in range(total - NBUF, total):
            scatters[j].wait()

    return k(uids2d, vids2d, user_table, video_table)


BLK = 4096


def _towers_body(xu_ref, xv_ref,
                 uw1_ref, ub1_ref, uw2_ref, ub2_ref,
                 vw1_ref, vb1_ref, vw2_ref, vb2_ref, out_ref):
    for t, (x_ref, w1_ref, b1_ref, w2_ref, b2_ref) in enumerate((
            (xu_ref, uw1_ref, ub1_ref, uw2_ref, ub2_ref),
            (xv_ref, vw1_ref, vb1_ref, vw2_ref, vb2_ref))):
        x = x_ref[...].astype(jnp.bfloat16)
        h = jnp.dot(x, w1_ref[...].astype(jnp.bfloat16),
                    preferred_element_type=jnp.float32)
        h = jnp.maximum(h + b1_ref[...], 0.0).astype(jnp.bfloat16)
        y = jnp.dot(h, w2_ref[...].astype(jnp.bfloat16),
                    preferred_element_type=jnp.float32)
        y = y + b2_ref[...]
        ss = jnp.sum(y * y, axis=1, keepdims=True)
        out_ref[t] = y * lax.rsqrt(jnp.maximum(ss, 1e-12))


def _tc_towers(emb, uW1, ub1, uW2, ub2, vW1, vb1, vW2, vb2):
    nblk = B // BLK
    wspec1 = pl.BlockSpec((D, H), lambda i: (0, 0))
    bspec1 = pl.BlockSpec((1, H), lambda i: (0, 0))
    wspec2 = pl.BlockSpec((H, D), lambda i: (0, 0))
    bspec2 = pl.BlockSpec((1, D), lambda i: (0, 0))
    return pl.pallas_call(
        _towers_body,
        grid=(nblk,),
        in_specs=[
            pl.BlockSpec((BLK, D), lambda i: (i, 0)),
            pl.BlockSpec((BLK, D), lambda i, n=nblk: (n + i, 0)),
            wspec1, bspec1, wspec2, bspec2,
            wspec1, bspec1, wspec2, bspec2,
        ],
        out_specs=pl.BlockSpec((2, BLK, D), lambda i: (0, i, 0)),
        out_shape=jax.ShapeDtypeStruct((2, B, D), jnp.float32),
    )(emb, emb, uW1, ub1, uW2, ub2, vW1, vb1, vW2, vb2)


def kernel(user_ids, video_ids, user_table, video_table,
           uW1, ub1, uW2, ub2, vW1, vb1, vW2, vb2):
    uids2d = user_ids.astype(jnp.int32).reshape(B // CHUNK, CHUNK)
    vids2d = video_ids.astype(jnp.int32).reshape(B // CHUNK, CHUNK)
    emb = _sc_gather(uids2d, vids2d, user_table, video_table)
    return _tc_towers(
        emb,
        uW1, ub1[None, :], uW2, ub2[None, :],
        vW1, vb1[None, :], vW2, vb2[None, :],
    )
